# baseline (speedup 1.0000x reference)
import functools

import jax
import jax.numpy as jnp
from jax import lax
from jax.experimental import pallas as pl
from jax.experimental.pallas import tpu as pltpu
from jax.experimental.pallas import tpu_sc as plsc

E = 64
D = 768
FF = 1536
N = 32768
CAP = 512
EC = E * CAP
ZROW = EC
D2 = D // 2

NB = 32
BN = N // NB

NC = 2
NS = 16
NW = NC * NS
TPW = N // NW
CHD = 128
CHC = 64


def _gating_body(x_ref, wg_ref, xs_ref, idx_ref, laux_ref, cnt_ref, me_ref,
                 tri_ref):
    i = pl.program_id(0)

    @pl.when(i == 0)
    def _():
        cnt_ref[...] = jnp.zeros_like(cnt_ref)
        me_ref[...] = jnp.zeros_like(me_ref)
        rr = lax.broadcasted_iota(jnp.int32, (BN, BN), 0)
        cc = lax.broadcasted_iota(jnp.int32, (BN, BN), 1)
        tri_ref[...] = (cc < rr).astype(jnp.bfloat16)

    xb = x_ref[...]
    logits = jnp.dot(xb, wg_ref[...], preferred_element_type=jnp.float32)
    rowmax = jnp.max(logits, axis=1, keepdims=True)
    ex = jnp.exp(logits - rowmax)
    sumex = jnp.sum(ex, axis=1, keepdims=True)
    gate = 1.0 / sumex
    lane = lax.broadcasted_iota(jnp.int32, (BN, E), 1)
    am = jnp.min(jnp.where(logits == rowmax, lane, E), axis=1, keepdims=True)
    mask = (lane == am).astype(jnp.float32)

    loc = jnp.dot(tri_ref[...], mask.astype(jnp.bfloat16),
                  preferred_element_type=jnp.float32) + cnt_ref[...]
    loc_s = jnp.sum(loc * mask, axis=1, keepdims=True)

    me_ref[...] += jnp.sum(ex / sumex, axis=0, keepdims=True)
    cnt_ref[...] += jnp.sum(mask, axis=0, keepdims=True)

    valid = loc_s < CAP
    slot = am * CAP + jnp.minimum(loc_s.astype(jnp.int32), CAP - 1)
    row = lax.broadcasted_iota(jnp.int32, (BN, 1), 0)
    idx_ref[0, :, :] = jnp.where(valid, slot, ZROW + (row & (CAP - 1)))
    xsf = xb * (gate * valid.astype(jnp.float32))
    v = lax.bitcast_convert_type(xsf, jnp.uint32)

    def _rnd(u):
        return (u + jnp.uint32(0x7FFF) + ((u >> 16) & jnp.uint32(1))) >> 16

    packed = (_rnd(v[:, D2:]) << 16) | _rnd(v[:, :D2])
    xs_ref[...] = lax.bitcast_convert_type(packed, jnp.int32)

    @pl.when(i == NB - 1)
    def _():
        laux_ref[...] = jnp.full((1, 1), E / (N * N)) * jnp.sum(
            me_ref[...] * cnt_ref[...])


_gating = pl.pallas_call(
    _gating_body,
    grid=(NB,),
    in_specs=[
        pl.BlockSpec((BN, D), lambda i: (i, 0)),
        pl.BlockSpec((D, E), lambda i: (0, 0)),
    ],
    out_specs=[
        pl.BlockSpec((BN, D2), lambda i: (i, 0)),
        pl.BlockSpec((1, BN, 1), lambda i: (i, 0, 0)),
        pl.BlockSpec((1, 1), lambda i: (0, 0)),
    ],
    out_shape=[
        jax.ShapeDtypeStruct((N, D2), jnp.int32),
        jax.ShapeDtypeStruct((NB, BN, 1), jnp.int32),
        jax.ShapeDtypeStruct((1, 1), jnp.float32),
    ],
    scratch_shapes=[
        pltpu.VMEM((1, E), jnp.float32),
        pltpu.VMEM((1, E), jnp.float32),
        pltpu.VMEM((BN, BN), jnp.bfloat16),
    ],
    compiler_params=pltpu.CompilerParams(
        dimension_semantics=("arbitrary",),
    ),
)


def _ffn_body(disp_ref, w1_ref, w2_ref, out_ref):
    e = pl.program_id(0)

    @pl.when(e < E)
    def _():
        a32 = lax.bitcast_convert_type(disp_ref[...], jnp.uint32)
        lo = lax.bitcast_convert_type(a32 << 16, jnp.float32)
        hi = lax.bitcast_convert_type(a32 & jnp.uint32(0xFFFF0000), jnp.float32)
        a = jnp.concatenate([lo, hi], axis=1).astype(jnp.bfloat16)
        h = jnp.maximum(
            jnp.dot(a, w1_ref[0].astype(jnp.bfloat16),
                    preferred_element_type=jnp.float32),
            0.0).astype(jnp.bfloat16)
        out_ref[...] = jnp.dot(h, w2_ref[0].astype(jnp.bfloat16),
                               preferred_element_type=jnp.float32)

    @pl.when(e == E)
    def _():
        out_ref[...] = jnp.zeros_like(out_ref)


_ffn = pl.pallas_call(
    _ffn_body,
    grid=(E + 1,),
    in_specs=[
        pl.BlockSpec((CAP, D2), lambda e: (e, 0)),
        pl.BlockSpec((1, D, FF), lambda e: (jnp.minimum(e, E - 1), 0, 0)),
        pl.BlockSpec((1, FF, D), lambda e: (jnp.minimum(e, E - 1), 0, 0)),
    ],
    out_specs=pl.BlockSpec((CAP, D), lambda e: (e, 0)),
    out_shape=jax.ShapeDtypeStruct((EC + CAP, D), jnp.float32),
    compiler_params=pltpu.CompilerParams(
        dimension_semantics=("arbitrary",),
        vmem_limit_bytes=100 * 1024 * 1024,
    ),
)


def _pipeline(nch, mk_in, mk_out):
    in_d = [None] * nch
    out_d = [None] * nch
    in_d[0] = mk_in(0)
    for j in range(nch):
        in_d[j].wait()
        out_d[j] = mk_out(j)
        if j + 1 < nch:
            if j >= 1:
                out_d[j - 1].wait()
            in_d[j + 1] = mk_in(j + 1)
    if nch >= 2:
        out_d[nch - 2].wait()
    out_d[nch - 1].wait()


@functools.lru_cache(maxsize=None)
def _sc_kernels():
    mesh = plsc.VectorSubcoreMesh(core_axis_name="c", subcore_axis_name="s")

    def _scratch(ch, width, dtype):
        return [
            pltpu.VMEM((TPW // ch, ch), jnp.int32),
            pltpu.VMEM((2, ch, width), dtype),
            pltpu.SemaphoreType.DMA,
            pltpu.SemaphoreType.DMA,
            pltpu.SemaphoreType.DMA,
        ]

    @functools.partial(
        pl.kernel,
        out_type=jax.ShapeDtypeStruct((EC + CAP, D2), jnp.int32),
        mesh=mesh,
        scratch_types=_scratch(CHD, D2, jnp.int32),
    )
    def dispatch(xs_hbm, idx_hbm, disp_hbm, idx_v, buf, in_sem, os0, os1):
        wid = lax.axis_index("s") * NC + lax.axis_index("c")
        pltpu.sync_copy(idx_hbm.at[wid], idx_v)
        out_sems = (os0, os1)

        def mk_in(j):
            return pltpu.async_copy(
                xs_hbm.at[pl.ds(wid * TPW + j * CHD, CHD)], buf.at[j % 2],
                in_sem)

        def mk_out(j):
            return pltpu.async_copy(
                buf.at[j % 2], disp_hbm.at[idx_v.at[j]], out_sems[j % 2])

        _pipeline(TPW // CHD, mk_in, mk_out)

    @functools.partial(
        pl.kernel,
        out_type=jax.ShapeDtypeStruct((N, D), jnp.float32),
        mesh=mesh,
        scratch_types=_scratch(CHC, D, jnp.float32),
    )
    def combine(eo_hbm, idx_hbm, y_hbm, idx_v, buf, in_sem, os0, os1):
        wid = lax.axis_index("s") * NC + lax.axis_index("c")
        pltpu.sync_copy(idx_hbm.at[wid], idx_v)
        out_sems = (os0, os1)

        def mk_in(j):
            return pltpu.async_copy(
                eo_hbm.at[idx_v.at[j]], buf.at[j % 2], in_sem)

        def mk_out(j):
            return pltpu.async_copy(
                buf.at[j % 2], y_hbm.at[pl.ds(wid * TPW + j * CHC, CHC)],
                out_sems[j % 2])

        _pipeline(TPW // CHC, mk_in, mk_out)

    return dispatch, combine


def kernel(x, wg, w1, w2):
    dispatch, combine = _sc_kernels()
    xs, idx3, laux = _gating(x, wg)
    disp = dispatch(xs, idx3.reshape(NW, TPW // CHD, CHD))
    eo = _ffn(disp, w1, w2)
    disp2 = dispatch(xs, idx3.reshape(NW, TPW // CHD, CHD))
    y = combine(eo, idx3.reshape(NW, TPW // CHC, CHC))
    return y, laux[0, 0] + 0.0 * disp2[0, 0].astype(jnp.float32)


# speedup vs baseline: 1.0227x; 1.0227x over previous
import functools

import jax
import jax.numpy as jnp
from jax import lax
from jax.experimental import pallas as pl
from jax.experimental.pallas import tpu as pltpu
from jax.experimental.pallas import tpu_sc as plsc

E = 64
D = 768
FF = 1536
N = 32768
CAP = 512
EC = E * CAP
ZROW = EC
D2 = D // 2

NB = 32
BN = N // NB

NC = 2
NS = 16
NW = NC * NS
TPW = N // NW
CHD = 128
CHC = 64


def _gating_body(x_ref, wg_ref, xs_ref, idx_ref, laux_ref, cnt_ref, me_ref,
                 tri_ref, triu_ref):
    i = pl.program_id(0)

    @pl.when(i == 0)
    def _():
        cnt_ref[...] = jnp.zeros_like(cnt_ref)
        me_ref[...] = jnp.zeros_like(me_ref)
        rr = lax.broadcasted_iota(jnp.int32, (BN, BN), 0)
        cc = lax.broadcasted_iota(jnp.int32, (BN, BN), 1)
        tri_ref[...] = (cc < rr).astype(jnp.bfloat16)
        ur = lax.broadcasted_iota(jnp.int32, (E, E), 0)
        uc = lax.broadcasted_iota(jnp.int32, (E, E), 1)
        triu_ref[...] = (ur < uc).astype(jnp.bfloat16)

    xb = x_ref[...]
    logits = jnp.dot(xb, wg_ref[...], preferred_element_type=jnp.float32)
    rowmax = jnp.max(logits, axis=1, keepdims=True)
    ex = jnp.exp(logits - rowmax)
    sumex = jnp.sum(ex, axis=1, keepdims=True)
    gate = 1.0 / sumex
    multi = (logits == rowmax).astype(jnp.bfloat16)
    excl = jnp.dot(multi, triu_ref[...], preferred_element_type=jnp.float32)
    mask = jnp.where(excl == 0.0, multi.astype(jnp.float32), 0.0)
    lanecol = lax.broadcasted_iota(jnp.int32, (E, 1), 0).astype(jnp.bfloat16)
    am = jnp.dot(mask.astype(jnp.bfloat16), lanecol,
                 preferred_element_type=jnp.float32).astype(jnp.int32)

    loc = jnp.dot(tri_ref[...], mask.astype(jnp.bfloat16),
                  preferred_element_type=jnp.float32) + cnt_ref[...]
    loc_s = jnp.sum(loc * mask, axis=1, keepdims=True)

    me_ref[...] += jnp.sum(ex / sumex, axis=0, keepdims=True)
    cnt_ref[...] += jnp.sum(mask, axis=0, keepdims=True)

    valid = loc_s < CAP
    slot = am * CAP + jnp.minimum(loc_s.astype(jnp.int32), CAP - 1)
    row = lax.broadcasted_iota(jnp.int32, (BN, 1), 0)
    idx_ref[0, :, :] = jnp.where(valid, slot, ZROW + (row & (CAP - 1)))
    xsf = xb * (gate * valid.astype(jnp.float32))
    v = lax.bitcast_convert_type(xsf, jnp.uint32) + jnp.uint32(0x8000)
    packed = (v[:, D2:] & jnp.uint32(0xFFFF0000)) | (v[:, :D2] >> 16)
    xs_ref[...] = lax.bitcast_convert_type(packed, jnp.int32)

    @pl.when(i == NB - 1)
    def _():
        laux_ref[...] = jnp.full((1, 1), E / (N * N)) * jnp.sum(
            me_ref[...] * cnt_ref[...])


_gating = pl.pallas_call(
    _gating_body,
    grid=(NB,),
    in_specs=[
        pl.BlockSpec((BN, D), lambda i: (i, 0)),
        pl.BlockSpec((D, E), lambda i: (0, 0)),
    ],
    out_specs=[
        pl.BlockSpec((BN, D2), lambda i: (i, 0)),
        pl.BlockSpec((1, BN, 1), lambda i: (i, 0, 0)),
        pl.BlockSpec((1, 1), lambda i: (0, 0)),
    ],
    out_shape=[
        jax.ShapeDtypeStruct((N, D2), jnp.int32),
        jax.ShapeDtypeStruct((NB, BN, 1), jnp.int32),
        jax.ShapeDtypeStruct((1, 1), jnp.float32),
    ],
    scratch_shapes=[
        pltpu.VMEM((1, E), jnp.float32),
        pltpu.VMEM((1, E), jnp.float32),
        pltpu.VMEM((BN, BN), jnp.bfloat16),
        pltpu.VMEM((E, E), jnp.bfloat16),
    ],
    compiler_params=pltpu.CompilerParams(
        dimension_semantics=("arbitrary",),
    ),
)


def _ffn_body(disp_ref, w1_ref, w2_ref, out_ref):
    e = pl.program_id(0)

    @pl.when(e < E)
    def _():
        a32 = lax.bitcast_convert_type(disp_ref[...], jnp.uint32)
        lo = lax.bitcast_convert_type(a32 << 16, jnp.float32)
        hi = lax.bitcast_convert_type(a32 & jnp.uint32(0xFFFF0000), jnp.float32)
        a = jnp.concatenate([lo, hi], axis=1).astype(jnp.bfloat16)
        h = jnp.maximum(
            jnp.dot(a, w1_ref[0].astype(jnp.bfloat16),
                    preferred_element_type=jnp.float32),
            0.0).astype(jnp.bfloat16)
        out_ref[...] = jnp.dot(h, w2_ref[0].astype(jnp.bfloat16),
                               preferred_element_type=jnp.float32)

    @pl.when(e == E)
    def _():
        out_ref[...] = jnp.zeros_like(out_ref)


_ffn = pl.pallas_call(
    _ffn_body,
    grid=(E + 1,),
    in_specs=[
        pl.BlockSpec((CAP, D2), lambda e: (e, 0)),
        pl.BlockSpec((1, D, FF), lambda e: (jnp.minimum(e, E - 1), 0, 0)),
        pl.BlockSpec((1, FF, D), lambda e: (jnp.minimum(e, E - 1), 0, 0)),
    ],
    out_specs=pl.BlockSpec((CAP, D), lambda e: (e, 0)),
    out_shape=jax.ShapeDtypeStruct((EC + CAP, D), jnp.float32),
    compiler_params=pltpu.CompilerParams(
        dimension_semantics=("arbitrary",),
        vmem_limit_bytes=100 * 1024 * 1024,
    ),
)


def _pipeline(nch, mk_in, mk_out):
    in_d = [None] * nch
    out_d = [None] * nch
    in_d[0] = mk_in(0)
    for j in range(nch):
        in_d[j].wait()
        out_d[j] = mk_out(j)
        if j + 1 < nch:
            if j >= 1:
                out_d[j - 1].wait()
            in_d[j + 1] = mk_in(j + 1)
    if nch >= 2:
        out_d[nch - 2].wait()
    out_d[nch - 1].wait()


@functools.lru_cache(maxsize=None)
def _sc_kernels():
    mesh = plsc.VectorSubcoreMesh(core_axis_name="c", subcore_axis_name="s")

    def _scratch(ch, width, dtype):
        return [
            pltpu.VMEM((TPW // ch, ch), jnp.int32),
            pltpu.VMEM((2, ch, width), dtype),
            pltpu.SemaphoreType.DMA,
            pltpu.SemaphoreType.DMA,
            pltpu.SemaphoreType.DMA,
        ]

    @functools.partial(
        pl.kernel,
        out_type=jax.ShapeDtypeStruct((EC + CAP, D2), jnp.int32),
        mesh=mesh,
        scratch_types=_scratch(CHD, D2, jnp.int32),
    )
    def dispatch(xs_hbm, idx_hbm, disp_hbm, idx_v, buf, in_sem, os0, os1):
        wid = lax.axis_index("s") * NC + lax.axis_index("c")
        pltpu.sync_copy(idx_hbm.at[wid], idx_v)
        out_sems = (os0, os1)

        def mk_in(j):
            return pltpu.async_copy(
                xs_hbm.at[pl.ds(wid * TPW + j * CHD, CHD)], buf.at[j % 2],
                in_sem)

        def mk_out(j):
            return pltpu.async_copy(
                buf.at[j % 2], disp_hbm.at[idx_v.at[j]], out_sems[j % 2])

        _pipeline(TPW // CHD, mk_in, mk_out)

    @functools.partial(
        pl.kernel,
        out_type=jax.ShapeDtypeStruct((N, D), jnp.float32),
        mesh=mesh,
        scratch_types=_scratch(CHC, D, jnp.float32),
    )
    def combine(eo_hbm, idx_hbm, y_hbm, idx_v, buf, in_sem, os0, os1):
        wid = lax.axis_index("s") * NC + lax.axis_index("c")
        pltpu.sync_copy(idx_hbm.at[wid], idx_v)
        out_sems = (os0, os1)

        def mk_in(j):
            return pltpu.async_copy(
                eo_hbm.at[idx_v.at[j]], buf.at[j % 2], in_sem)

        def mk_out(j):
            return pltpu.async_copy(
                buf.at[j % 2], y_hbm.at[pl.ds(wid * TPW + j * CHC, CHC)],
                out_sems[j % 2])

        _pipeline(TPW // CHC, mk_in, mk_out)

    return dispatch, combine


def kernel(x, wg, w1, w2):
    dispatch, combine = _sc_kernels()
    xs, idx3, laux = _gating(x, wg)
    disp = dispatch(xs, idx3.reshape(NW, TPW // CHD, CHD))
    eo = _ffn(disp, w1, w2)
    y = combine(eo, idx3.reshape(NW, TPW // CHC, CHC))
    return y, laux[0, 0]
